# Initial kernel scaffold; baseline (speedup 1.0000x reference)
import jax
import jax.numpy as jnp
from jax.experimental import pallas as pl


def kernel(h, segment_ids, fc1_w, fc1_b, fc2_w, fc2_b):
    raise NotImplementedError("write your pallas kernel here")


# speedup vs baseline: 10.6638x; 10.6638x over previous
import jax
import jax.numpy as jnp
from jax import lax
from jax.experimental import pallas as pl
from jax.experimental.pallas import tpu as pltpu

N = 100000
G = 256
D_IN = 128
D_H = 64
H = 4
B = 2000
NB = N // B

_NEG = -1e30


def _k1_body(h_ref, w1t_ref, b1_ref, w2t_ref, b2_ref, scores_ref, gm_ref, gm_s):
    i = pl.program_id(0)
    hb = h_ref[...]
    z = jnp.tanh(jnp.dot(hb, w1t_ref[...], preferred_element_type=jnp.float32)
                 + b1_ref[...])
    scores = jnp.dot(z, w2t_ref[...], preferred_element_type=jnp.float32) + b2_ref[...]
    scores_ref[...] = scores

    @pl.when(i == 0)
    def _():
        gm_s[...] = jnp.full((1, H), _NEG, jnp.float32)

    gm_s[...] = jnp.maximum(gm_s[...], jnp.max(scores, axis=0, keepdims=True))

    @pl.when(i == NB - 1)
    def _():
        gm_ref[...] = gm_s[...]


def _onehot(seg):
    return (seg[:, None] == lax.broadcasted_iota(jnp.int32, (B, G), 1)).astype(
        jnp.float32)


def _k2a_body(scores_ref, seg_ref, gm_ref, ssum_ref, acc_s):
    i = pl.program_id(0)
    seg = seg_ref[0, 0, :]
    e = jnp.exp(scores_ref[...] - gm_ref[...])
    oh = _onehot(seg)

    @pl.when(i == 0)
    def _():
        acc_s[...] = jnp.zeros((G, H), jnp.float32)

    acc_s[...] += lax.dot_general(oh, e, (((0,), (0,)), ((), ())),
                                  preferred_element_type=jnp.float32)

    @pl.when(i == NB - 1)
    def _():
        ssum_ref[...] = acc_s[...]


def _k2b_body(h_ref, scores_ref, seg_ref, gm_ref, ssum_ref, out_ref, acc_s):
    i = pl.program_id(0)
    seg = seg_ref[0, 0, :]
    e = jnp.exp(scores_ref[...] - gm_ref[...])
    oh = _onehot(seg)
    denom = jnp.dot(oh, ssum_ref[...], preferred_element_type=jnp.float32)
    w = jnp.sum(e / (denom + 1e-12), axis=1, keepdims=True) * (1.0 / H)
    wh = h_ref[...] * w

    @pl.when(i == 0)
    def _():
        acc_s[...] = jnp.zeros((G, D_IN), jnp.float32)

    acc_s[...] += lax.dot_general(oh, wh, (((0,), (0,)), ((), ())),
                                  preferred_element_type=jnp.float32)

    @pl.when(i == NB - 1)
    def _():
        out_ref[...] = acc_s[...]


def kernel(h, segment_ids, fc1_w, fc1_b, fc2_w, fc2_b):
    seg3d = segment_ids.astype(jnp.int32).reshape(NB, 1, B)
    w1t = fc1_w.T
    w2t = fc2_w.T
    b1 = fc1_b.reshape(1, D_H)
    b2 = fc2_b.reshape(1, H)

    scores, gm = pl.pallas_call(
        _k1_body,
        grid=(NB,),
        in_specs=[
            pl.BlockSpec((B, D_IN), lambda i: (i, 0)),
            pl.BlockSpec((D_IN, D_H), lambda i: (0, 0)),
            pl.BlockSpec((1, D_H), lambda i: (0, 0)),
            pl.BlockSpec((D_H, H), lambda i: (0, 0)),
            pl.BlockSpec((1, H), lambda i: (0, 0)),
        ],
        out_specs=[
            pl.BlockSpec((B, H), lambda i: (i, 0)),
            pl.BlockSpec((1, H), lambda i: (0, 0)),
        ],
        out_shape=[
            jax.ShapeDtypeStruct((N, H), jnp.float32),
            jax.ShapeDtypeStruct((1, H), jnp.float32),
        ],
        scratch_shapes=[pltpu.VMEM((1, H), jnp.float32)],
    )(h, w1t, b1, w2t, b2)

    ssum = pl.pallas_call(
        _k2a_body,
        grid=(NB,),
        in_specs=[
            pl.BlockSpec((B, H), lambda i: (i, 0)),
            pl.BlockSpec((1, 1, B), lambda i: (i, 0, 0)),
            pl.BlockSpec((1, H), lambda i: (0, 0)),
        ],
        out_specs=pl.BlockSpec((G, H), lambda i: (0, 0)),
        out_shape=jax.ShapeDtypeStruct((G, H), jnp.float32),
        scratch_shapes=[pltpu.VMEM((G, H), jnp.float32)],
    )(scores, seg3d, gm)

    out = pl.pallas_call(
        _k2b_body,
        grid=(NB,),
        in_specs=[
            pl.BlockSpec((B, D_IN), lambda i: (i, 0)),
            pl.BlockSpec((B, H), lambda i: (i, 0)),
            pl.BlockSpec((1, 1, B), lambda i: (i, 0, 0)),
            pl.BlockSpec((1, H), lambda i: (0, 0)),
            pl.BlockSpec((G, H), lambda i: (0, 0)),
        ],
        out_specs=pl.BlockSpec((G, D_IN), lambda i: (0, 0)),
        out_shape=jax.ShapeDtypeStruct((G, D_IN), jnp.float32),
        scratch_shapes=[pltpu.VMEM((G, D_IN), jnp.float32)],
    )(h, scores, seg3d, gm, ssum)

    return out


# speedup vs baseline: 11.5361x; 1.0818x over previous
import jax
import jax.numpy as jnp
from jax import lax
from jax.experimental import pallas as pl
from jax.experimental.pallas import tpu as pltpu

N = 100000
G = 256
D_IN = 128
D_H = 64
H = 4
B = 2000
NB = N // B

_NEG = -1e30


def _k1_body(h_ref, w1t_ref, b1_ref, w2t_ref, b2_ref, scores_ref, gm_ref, gm_s):
    i = pl.program_id(0)
    hb = h_ref[...]
    z = jnp.tanh(jnp.dot(hb, w1t_ref[...], preferred_element_type=jnp.float32)
                 + b1_ref[...])
    scores = jnp.dot(z, w2t_ref[...], preferred_element_type=jnp.float32) + b2_ref[...]
    scores_ref[...] = scores

    @pl.when(i == 0)
    def _():
        gm_s[...] = jnp.full((1, H), _NEG, jnp.float32)

    gm_s[...] = jnp.maximum(gm_s[...], jnp.max(scores, axis=0, keepdims=True))

    @pl.when(i == NB - 1)
    def _():
        gm_ref[...] = gm_s[...]


def _k2a_body(scores_ref, seg_ref, gm_ref, ssum_ref, acc_s):
    i = pl.program_id(0)
    seg = seg_ref[0, 0, :]
    oh = (seg[:, None] == lax.broadcasted_iota(jnp.int32, (B, G), 1)).astype(
        jnp.float32)
    e = jnp.exp(scores_ref[...] - gm_ref[...])

    @pl.when(i == 0)
    def _():
        acc_s[...] = jnp.zeros((H, G), jnp.float32)

    acc_s[...] += lax.dot_general(e, oh, (((0,), (0,)), ((), ())),
                                  preferred_element_type=jnp.float32)

    @pl.when(i == NB - 1)
    def _():
        ssum_ref[...] = acc_s[...]


W = 72


def _k2b_body(h_ref, scores_ref, seg_ref, gm_ref, ssum_ref, out_ref, acc_s):
    i = pl.program_id(0)
    seg = seg_ref[0, 0, :]
    e = jnp.exp(scores_ref[...] - gm_ref[...])
    g0 = jnp.minimum(seg_ref[0, 0, 0] & ~7, G - W)
    span_ok = (seg_ref[0, 0, B - 1] - g0) < W

    @pl.when(i == 0)
    def _():
        acc_s[...] = jnp.zeros((G, D_IN), jnp.float32)

    @pl.when(span_ok)
    def _():
        rel = seg - g0
        ohw = (rel[:, None] == lax.broadcasted_iota(jnp.int32, (B, W), 1)
               ).astype(jnp.float32)
        denom = jnp.dot(ohw, ssum_ref[pl.ds(g0, W), :],
                        preferred_element_type=jnp.float32)
        w = jnp.sum(e / (denom + 1e-12), axis=1, keepdims=True) * (1.0 / H)
        wh = (h_ref[...] * w).astype(jnp.bfloat16)
        acc_s[pl.ds(g0, W), :] += lax.dot_general(
            ohw.astype(jnp.bfloat16), wh, (((0,), (0,)), ((), ())),
            preferred_element_type=jnp.float32)

    @pl.when(jnp.logical_not(span_ok))
    def _():
        oh = (seg[:, None] == lax.broadcasted_iota(jnp.int32, (B, G), 1)
              ).astype(jnp.float32)
        denom = jnp.dot(oh, ssum_ref[...], preferred_element_type=jnp.float32)
        w = jnp.sum(e / (denom + 1e-12), axis=1, keepdims=True) * (1.0 / H)
        wh = (h_ref[...] * w).astype(jnp.bfloat16)
        acc_s[...] += lax.dot_general(
            oh.astype(jnp.bfloat16), wh, (((0,), (0,)), ((), ())),
            preferred_element_type=jnp.float32)

    @pl.when(i == NB - 1)
    def _():
        out_ref[...] = acc_s[...]


def kernel(h, segment_ids, fc1_w, fc1_b, fc2_w, fc2_b):
    seg3d = segment_ids.astype(jnp.int32).reshape(NB, 1, B)
    w1t = fc1_w.T
    w2t = fc2_w.T
    b1 = fc1_b.reshape(1, D_H)
    b2 = fc2_b.reshape(1, H)

    scores, gm = pl.pallas_call(
        _k1_body,
        grid=(NB,),
        in_specs=[
            pl.BlockSpec((B, D_IN), lambda i: (i, 0)),
            pl.BlockSpec((D_IN, D_H), lambda i: (0, 0)),
            pl.BlockSpec((1, D_H), lambda i: (0, 0)),
            pl.BlockSpec((D_H, H), lambda i: (0, 0)),
            pl.BlockSpec((1, H), lambda i: (0, 0)),
        ],
        out_specs=[
            pl.BlockSpec((B, H), lambda i: (i, 0)),
            pl.BlockSpec((1, H), lambda i: (0, 0)),
        ],
        out_shape=[
            jax.ShapeDtypeStruct((N, H), jnp.float32),
            jax.ShapeDtypeStruct((1, H), jnp.float32),
        ],
        scratch_shapes=[pltpu.VMEM((1, H), jnp.float32)],
    )(h, w1t, b1, w2t, b2)

    ssum = pl.pallas_call(
        _k2a_body,
        grid=(NB,),
        in_specs=[
            pl.BlockSpec((B, H), lambda i: (i, 0)),
            pl.BlockSpec((1, 1, B), lambda i: (i, 0, 0)),
            pl.BlockSpec((1, H), lambda i: (0, 0)),
        ],
        out_specs=pl.BlockSpec((H, G), lambda i: (0, 0)),
        out_shape=jax.ShapeDtypeStruct((H, G), jnp.float32),
        scratch_shapes=[pltpu.VMEM((H, G), jnp.float32)],
    )(scores, seg3d, gm)

    ssum = ssum.T

    out = pl.pallas_call(
        _k2b_body,
        grid=(NB,),
        in_specs=[
            pl.BlockSpec((B, D_IN), lambda i: (i, 0)),
            pl.BlockSpec((B, H), lambda i: (i, 0)),
            pl.BlockSpec((1, 1, B), lambda i: (i, 0, 0)),
            pl.BlockSpec((1, H), lambda i: (0, 0)),
            pl.BlockSpec((G, H), lambda i: (0, 0)),
        ],
        out_specs=pl.BlockSpec((G, D_IN), lambda i: (0, 0)),
        out_shape=jax.ShapeDtypeStruct((G, D_IN), jnp.float32),
        scratch_shapes=[pltpu.VMEM((G, D_IN), jnp.float32)],
    )(h, scores, seg3d, gm, ssum)

    return out


# speedup vs baseline: 13.5697x; 1.1763x over previous
import jax
import jax.numpy as jnp
from jax import lax
from jax.experimental import pallas as pl
from jax.experimental.pallas import tpu as pltpu

N = 100000
G = 256
D_IN = 128
D_H = 64
H = 4
B = 2000
NB = N // B
W = 72

_NEG = -1e30


def _window(seg_ref):
    g0 = jnp.minimum(seg_ref[0, 0, 0] & ~7, G - W)
    span_ok = (seg_ref[0, 0, B - 1] - g0) < W
    return g0, span_ok


def _k1_body(h_ref, w1t_ref, b1_ref, w2t_ref, seg_ref,
             scores_ref, gm_ref, rssum_ref, gm_s, acc_s):
    i = pl.program_id(0)
    hb = h_ref[...]
    z = jnp.tanh(jnp.dot(hb, w1t_ref[...], preferred_element_type=jnp.float32)
                 + b1_ref[...])
    scores = jnp.dot(z, w2t_ref[...], preferred_element_type=jnp.float32)
    scores_ref[...] = scores

    @pl.when(i == 0)
    def _():
        gm_s[...] = jnp.full((1, H), _NEG, jnp.float32)
        acc_s[...] = jnp.zeros((G, H), jnp.float32)

    gm_old = gm_s[...]
    gm_new = jnp.maximum(gm_old, jnp.max(scores, axis=0, keepdims=True))
    gm_s[...] = gm_new
    acc_s[...] *= jnp.exp(gm_old - gm_new)
    e = jnp.exp(scores - gm_new)

    seg = seg_ref[0, 0, :]
    g0, span_ok = _window(seg_ref)

    @pl.when(span_ok)
    def _():
        rel = seg - g0
        ohw = (rel[:, None] == lax.broadcasted_iota(jnp.int32, (B, W), 1)
               ).astype(jnp.float32)
        acc_s[pl.ds(g0, W), :] += lax.dot_general(
            ohw, e, (((0,), (0,)), ((), ())),
            preferred_element_type=jnp.float32)

    @pl.when(jnp.logical_not(span_ok))
    def _():
        oh = (seg[:, None] == lax.broadcasted_iota(jnp.int32, (B, G), 1)
              ).astype(jnp.float32)
        acc_s[...] += lax.dot_general(
            oh, e, (((0,), (0,)), ((), ())),
            preferred_element_type=jnp.float32)

    @pl.when(i == NB - 1)
    def _():
        gm_ref[...] = gm_s[...]
        rssum_ref[...] = 1.0 / (acc_s[...] + 1e-12)


def _k2_body(h_ref, scores_ref, seg_ref, gm_ref, rssum_ref, out_ref, acc_s):
    i = pl.program_id(0)
    seg = seg_ref[0, 0, :]
    e = jnp.exp(scores_ref[...] - gm_ref[...])
    ones = jnp.full((H, 1), 1.0 / H, jnp.float32)
    g0, span_ok = _window(seg_ref)

    @pl.when(i == 0)
    def _():
        acc_s[...] = jnp.zeros((G, D_IN), jnp.float32)

    @pl.when(span_ok)
    def _():
        rel = seg - g0
        ohw = (rel[:, None] == lax.broadcasted_iota(jnp.int32, (B, W), 1)
               ).astype(jnp.float32)
        rdenom = jnp.dot(ohw, rssum_ref[pl.ds(g0, W), :],
                         preferred_element_type=jnp.float32)
        w = jnp.dot(e * rdenom, ones, preferred_element_type=jnp.float32)
        wh = (h_ref[...] * w).astype(jnp.bfloat16)
        acc_s[pl.ds(g0, W), :] += lax.dot_general(
            ohw.astype(jnp.bfloat16), wh, (((0,), (0,)), ((), ())),
            preferred_element_type=jnp.float32)

    @pl.when(jnp.logical_not(span_ok))
    def _():
        oh = (seg[:, None] == lax.broadcasted_iota(jnp.int32, (B, G), 1)
              ).astype(jnp.float32)
        rdenom = jnp.dot(oh, rssum_ref[...], preferred_element_type=jnp.float32)
        w = jnp.dot(e * rdenom, ones, preferred_element_type=jnp.float32)
        wh = (h_ref[...] * w).astype(jnp.bfloat16)
        acc_s[...] += lax.dot_general(
            oh.astype(jnp.bfloat16), wh, (((0,), (0,)), ((), ())),
            preferred_element_type=jnp.float32)

    @pl.when(i == NB - 1)
    def _():
        out_ref[...] = acc_s[...]


def kernel(h, segment_ids, fc1_w, fc1_b, fc2_w, fc2_b):
    seg3d = segment_ids.astype(jnp.int32).reshape(NB, 1, B)
    w1t = fc1_w.T
    w2t = fc2_w.T
    b1 = fc1_b.reshape(1, D_H)

    scores, gm, rssum = pl.pallas_call(
        _k1_body,
        grid=(NB,),
        in_specs=[
            pl.BlockSpec((B, D_IN), lambda i: (i, 0)),
            pl.BlockSpec((D_IN, D_H), lambda i: (0, 0)),
            pl.BlockSpec((1, D_H), lambda i: (0, 0)),
            pl.BlockSpec((D_H, H), lambda i: (0, 0)),
            pl.BlockSpec((1, 1, B), lambda i: (i, 0, 0)),
        ],
        out_specs=[
            pl.BlockSpec((B, H), lambda i: (i, 0)),
            pl.BlockSpec((1, H), lambda i: (0, 0)),
            pl.BlockSpec((G, H), lambda i: (0, 0)),
        ],
        out_shape=[
            jax.ShapeDtypeStruct((N, H), jnp.float32),
            jax.ShapeDtypeStruct((1, H), jnp.float32),
            jax.ShapeDtypeStruct((G, H), jnp.float32),
        ],
        scratch_shapes=[pltpu.VMEM((1, H), jnp.float32),
                        pltpu.VMEM((G, H), jnp.float32)],
    )(h, w1t, b1, w2t, seg3d)

    out = pl.pallas_call(
        _k2_body,
        grid=(NB,),
        in_specs=[
            pl.BlockSpec((B, D_IN), lambda i: (i, 0)),
            pl.BlockSpec((B, H), lambda i: (i, 0)),
            pl.BlockSpec((1, 1, B), lambda i: (i, 0, 0)),
            pl.BlockSpec((1, H), lambda i: (0, 0)),
            pl.BlockSpec((G, H), lambda i: (0, 0)),
        ],
        out_specs=pl.BlockSpec((G, D_IN), lambda i: (0, 0)),
        out_shape=jax.ShapeDtypeStruct((G, D_IN), jnp.float32),
        scratch_shapes=[pltpu.VMEM((G, D_IN), jnp.float32)],
    )(h, scores, seg3d, gm, rssum)

    return out


# speedup vs baseline: 13.6365x; 1.0049x over previous
import jax
import jax.numpy as jnp
from jax import lax
from jax.experimental import pallas as pl
from jax.experimental.pallas import tpu as pltpu

N = 100000
G = 256
D_IN = 128
D_H = 64
H = 4
B = 2000
NB = N // B
W = 72

_NEG = -1e30


def _window(seg_ref):
    g0 = jnp.minimum(seg_ref[0, 0, 0] & ~7, G - W)
    span_ok = (seg_ref[0, 0, B - 1] - g0) < W
    return g0, span_ok


def _k1_body(h_ref, w1t_ref, b1_ref, w2t_ref, seg_ref,
             scores_ref, gm_ref, rssum_ref, gm_s, acc_s):
    i = pl.program_id(0)
    hb = h_ref[...]
    z = jnp.tanh(jnp.dot(hb, w1t_ref[...], preferred_element_type=jnp.float32)
                 + b1_ref[...])
    scores = jnp.dot(z, w2t_ref[...], preferred_element_type=jnp.float32)
    st = lax.dot_general(w2t_ref[...], z, (((0,), (1,)), ((), ())),
                         preferred_element_type=jnp.float32)
    scores_ref[...] = st.reshape(1, H, B)

    @pl.when(i == 0)
    def _():
        gm_s[...] = jnp.full((1, H), _NEG, jnp.float32)
        acc_s[...] = jnp.zeros((G, H), jnp.float32)

    gm_old = gm_s[...]
    gm_new = jnp.maximum(gm_old, jnp.max(scores, axis=0, keepdims=True))
    gm_s[...] = gm_new
    acc_s[...] *= jnp.exp(gm_old - gm_new)
    e = jnp.exp(scores - gm_new)

    seg = seg_ref[0, 0, :]
    g0, span_ok = _window(seg_ref)

    @pl.when(span_ok)
    def _():
        rel = seg - g0
        ohw = (rel[:, None] == lax.broadcasted_iota(jnp.int32, (B, W), 1)
               ).astype(jnp.float32)
        acc_s[pl.ds(g0, W), :] += lax.dot_general(
            ohw, e, (((0,), (0,)), ((), ())),
            preferred_element_type=jnp.float32)

    @pl.when(jnp.logical_not(span_ok))
    def _():
        oh = (seg[:, None] == lax.broadcasted_iota(jnp.int32, (B, G), 1)
              ).astype(jnp.float32)
        acc_s[...] += lax.dot_general(
            oh, e, (((0,), (0,)), ((), ())),
            preferred_element_type=jnp.float32)

    @pl.when(i == NB - 1)
    def _():
        gm_ref[...] = gm_s[...]
        rssum_ref[...] = 1.0 / (acc_s[...] + 1e-12)


def _k2_body(h_ref, scores_ref, seg_ref, gm_ref, rssum_ref, out_ref, acc_s):
    i = pl.program_id(0)
    seg = seg_ref[0, 0, :]
    e = jnp.exp(jnp.transpose(scores_ref[...][0]) - gm_ref[...])
    ones = jnp.full((H, 1), 1.0 / H, jnp.float32)
    g0, span_ok = _window(seg_ref)

    @pl.when(i == 0)
    def _():
        acc_s[...] = jnp.zeros((G, D_IN), jnp.float32)

    @pl.when(span_ok)
    def _():
        rel = seg - g0
        ohw = (rel[:, None] == lax.broadcasted_iota(jnp.int32, (B, W), 1)
               ).astype(jnp.float32)
        rdenom = jnp.dot(ohw, rssum_ref[pl.ds(g0, W), :],
                         preferred_element_type=jnp.float32)
        w = jnp.dot(e * rdenom, ones, preferred_element_type=jnp.float32)
        wh = (h_ref[...] * w).astype(jnp.bfloat16)
        acc_s[pl.ds(g0, W), :] += lax.dot_general(
            ohw.astype(jnp.bfloat16), wh, (((0,), (0,)), ((), ())),
            preferred_element_type=jnp.float32)

    @pl.when(jnp.logical_not(span_ok))
    def _():
        oh = (seg[:, None] == lax.broadcasted_iota(jnp.int32, (B, G), 1)
              ).astype(jnp.float32)
        rdenom = jnp.dot(oh, rssum_ref[...], preferred_element_type=jnp.float32)
        w = jnp.dot(e * rdenom, ones, preferred_element_type=jnp.float32)
        wh = (h_ref[...] * w).astype(jnp.bfloat16)
        acc_s[...] += lax.dot_general(
            oh.astype(jnp.bfloat16), wh, (((0,), (0,)), ((), ())),
            preferred_element_type=jnp.float32)

    @pl.when(i == NB - 1)
    def _():
        out_ref[...] = acc_s[...]


def kernel(h, segment_ids, fc1_w, fc1_b, fc2_w, fc2_b):
    seg3d = segment_ids.astype(jnp.int32).reshape(NB, 1, B)
    w1t = fc1_w.T
    w2t = fc2_w.T
    b1 = fc1_b.reshape(1, D_H)

    scores, gm, rssum = pl.pallas_call(
        _k1_body,
        grid=(NB,),
        in_specs=[
            pl.BlockSpec((B, D_IN), lambda i: (i, 0)),
            pl.BlockSpec((D_IN, D_H), lambda i: (0, 0)),
            pl.BlockSpec((1, D_H), lambda i: (0, 0)),
            pl.BlockSpec((D_H, H), lambda i: (0, 0)),
            pl.BlockSpec((1, 1, B), lambda i: (i, 0, 0)),
        ],
        out_specs=[
            pl.BlockSpec((1, H, B), lambda i: (i, 0, 0)),
            pl.BlockSpec((1, H), lambda i: (0, 0)),
            pl.BlockSpec((G, H), lambda i: (0, 0)),
        ],
        out_shape=[
            jax.ShapeDtypeStruct((NB, H, B), jnp.float32),
            jax.ShapeDtypeStruct((1, H), jnp.float32),
            jax.ShapeDtypeStruct((G, H), jnp.float32),
        ],
        scratch_shapes=[pltpu.VMEM((1, H), jnp.float32),
                        pltpu.VMEM((G, H), jnp.float32)],
    )(h, w1t, b1, w2t, seg3d)

    out = pl.pallas_call(
        _k2_body,
        grid=(NB,),
        in_specs=[
            pl.BlockSpec((B, D_IN), lambda i: (i, 0)),
            pl.BlockSpec((1, H, B), lambda i: (i, 0, 0)),
            pl.BlockSpec((1, 1, B), lambda i: (i, 0, 0)),
            pl.BlockSpec((1, H), lambda i: (0, 0)),
            pl.BlockSpec((G, H), lambda i: (0, 0)),
        ],
        out_specs=pl.BlockSpec((G, D_IN), lambda i: (0, 0)),
        out_shape=jax.ShapeDtypeStruct((G, D_IN), jnp.float32),
        scratch_shapes=[pltpu.VMEM((G, D_IN), jnp.float32)],
    )(h, scores, seg3d, gm, rssum)

    return out


# speedup vs baseline: 18.6016x; 1.3641x over previous
import jax
import jax.numpy as jnp
from jax import lax
from jax.experimental import pallas as pl
from jax.experimental.pallas import tpu as pltpu

N = 100000
G = 256
D_IN = 128
D_H = 64
H = 4
B = 2000
NB = N // B
W = 72

_NEG = -1e30


def _window(seg_ref):
    g0 = jnp.minimum(seg_ref[0, 0, 0] & ~7, G - W)
    span_ok = (seg_ref[0, 0, B - 1] - g0) < W
    return g0, span_ok


def _ohT(seg_row, g0, width):
    return (lax.broadcasted_iota(jnp.int32, (width, B), 0) + g0
            == seg_row).astype(jnp.float32)


def _k1_body(h_ref, w1t_ref, b1_ref, w2t_ref, seg_ref,
             scores_ref, gm_ref, rssum_ref, gm_s, acc_s):
    i = pl.program_id(0)
    z = jnp.tanh(jnp.dot(h_ref[...], w1t_ref[...],
                         preferred_element_type=jnp.float32) + b1_ref[...])
    st = lax.dot_general(w2t_ref[...], z, (((0,), (1,)), ((), ())),
                         preferred_element_type=jnp.float32)
    scores_ref[...] = st.reshape(1, H, B)

    @pl.when(i == 0)
    def _():
        gm_s[...] = jnp.full((H, 1), _NEG, jnp.float32)
        acc_s[...] = jnp.zeros((G, H), jnp.float32)

    gm_old = gm_s[...]
    gm_new = jnp.maximum(gm_old, jnp.max(st, axis=1, keepdims=True))
    gm_s[...] = gm_new
    i4 = (lax.broadcasted_iota(jnp.int32, (H, H), 0)
          == lax.broadcasted_iota(jnp.int32, (H, H), 1)).astype(jnp.float32)
    fac = lax.dot_general(jnp.exp(gm_old - gm_new), i4,
                          (((0,), (0,)), ((), ())),
                          preferred_element_type=jnp.float32)
    acc_s[...] *= fac
    e_t = jnp.exp(st - gm_new)

    seg_row = seg_ref[0, :, :]
    g0, span_ok = _window(seg_ref)

    @pl.when(span_ok)
    def _():
        ohT = _ohT(seg_row, g0, W)
        ps = lax.dot_general(e_t, ohT, (((1,), (1,)), ((), ())),
                             preferred_element_type=jnp.float32)
        acc_s[pl.ds(g0, W), :] += jnp.transpose(ps)

    @pl.when(jnp.logical_not(span_ok))
    def _():
        ohT = _ohT(seg_row, 0, G)
        ps = lax.dot_general(e_t, ohT, (((1,), (1,)), ((), ())),
                             preferred_element_type=jnp.float32)
        acc_s[...] += jnp.transpose(ps)

    @pl.when(i == NB - 1)
    def _():
        gm_ref[...] = gm_s[...]
        rssum_ref[...] = 1.0 / (acc_s[...] + 1e-12)


def _k2_body(h_ref, scores_ref, seg_ref, gm_ref, rssum_ref, out_ref, acc_s):
    i = pl.program_id(0)
    e_t = jnp.exp(scores_ref[...][0] - gm_ref[...])
    seg_row = seg_ref[0, :, :]
    g0, span_ok = _window(seg_ref)

    @pl.when(i == 0)
    def _():
        acc_s[...] = jnp.zeros((G, D_IN), jnp.float32)

    hb = h_ref[...].astype(jnp.bfloat16)

    @pl.when(span_ok)
    def _():
        ohT = _ohT(seg_row, g0, W)
        rd_t = lax.dot_general(rssum_ref[pl.ds(g0, W), :], ohT,
                               (((0,), (0,)), ((), ())),
                               preferred_element_type=jnp.float32)
        w_t = jnp.sum(e_t * rd_t, axis=0, keepdims=True) * (1.0 / H)
        ohTw = ohT.astype(jnp.bfloat16) * w_t.astype(jnp.bfloat16)
        acc_s[pl.ds(g0, W), :] += lax.dot_general(
            ohTw, hb, (((1,), (0,)), ((), ())),
            preferred_element_type=jnp.float32)

    @pl.when(jnp.logical_not(span_ok))
    def _():
        ohT = _ohT(seg_row, 0, G)
        rd_t = lax.dot_general(rssum_ref[...], ohT,
                               (((0,), (0,)), ((), ())),
                               preferred_element_type=jnp.float32)
        w_t = jnp.sum(e_t * rd_t, axis=0, keepdims=True) * (1.0 / H)
        ohTw = ohT.astype(jnp.bfloat16) * w_t.astype(jnp.bfloat16)
        acc_s[...] += lax.dot_general(
            ohTw, hb, (((1,), (0,)), ((), ())),
            preferred_element_type=jnp.float32)

    @pl.when(i == NB - 1)
    def _():
        out_ref[...] = acc_s[...]


def kernel(h, segment_ids, fc1_w, fc1_b, fc2_w, fc2_b):
    seg3d = segment_ids.astype(jnp.int32).reshape(NB, 1, B)
    w1t = fc1_w.T
    w2t = fc2_w.T
    b1 = fc1_b.reshape(1, D_H)

    scores, gm, rssum = pl.pallas_call(
        _k1_body,
        grid=(NB,),
        in_specs=[
            pl.BlockSpec((B, D_IN), lambda i: (i, 0)),
            pl.BlockSpec((D_IN, D_H), lambda i: (0, 0)),
            pl.BlockSpec((1, D_H), lambda i: (0, 0)),
            pl.BlockSpec((D_H, H), lambda i: (0, 0)),
            pl.BlockSpec((1, 1, B), lambda i: (i, 0, 0)),
        ],
        out_specs=[
            pl.BlockSpec((1, H, B), lambda i: (i, 0, 0)),
            pl.BlockSpec((H, 1), lambda i: (0, 0)),
            pl.BlockSpec((G, H), lambda i: (0, 0)),
        ],
        out_shape=[
            jax.ShapeDtypeStruct((NB, H, B), jnp.float32),
            jax.ShapeDtypeStruct((H, 1), jnp.float32),
            jax.ShapeDtypeStruct((G, H), jnp.float32),
        ],
        scratch_shapes=[pltpu.VMEM((H, 1), jnp.float32),
                        pltpu.VMEM((G, H), jnp.float32)],
    )(h, w1t, b1, w2t, seg3d)

    out = pl.pallas_call(
        _k2_body,
        grid=(NB,),
        in_specs=[
            pl.BlockSpec((B, D_IN), lambda i: (i, 0)),
            pl.BlockSpec((1, H, B), lambda i: (i, 0, 0)),
            pl.BlockSpec((1, 1, B), lambda i: (i, 0, 0)),
            pl.BlockSpec((H, 1), lambda i: (0, 0)),
            pl.BlockSpec((G, H), lambda i: (0, 0)),
        ],
        out_specs=pl.BlockSpec((G, D_IN), lambda i: (0, 0)),
        out_shape=jax.ShapeDtypeStruct((G, D_IN), jnp.float32),
        scratch_shapes=[pltpu.VMEM((G, D_IN), jnp.float32)],
    )(h, scores, seg3d, gm, rssum)

    return out


# speedup vs baseline: 32.3208x; 1.7375x over previous
import jax
import jax.numpy as jnp
from jax import lax
from jax.experimental import pallas as pl
from jax.experimental.pallas import tpu as pltpu

N = 100000
G = 256
D_IN = 128
D_H = 64
H = 4
B = 4000
NB = N // B
W = 72

_NEG = -1e30


def _window(seg_ref):
    g0 = jnp.minimum(seg_ref[0, 0, 0] & ~7, G - W)
    span_ok = (seg_ref[0, 0, B - 1] - g0) < W
    return g0, span_ok


def _ohT(seg_row, g0, width):
    return (lax.broadcasted_iota(jnp.int32, (width, B), 0) + g0
            == seg_row).astype(jnp.float32)


def _body(h_ref, w1t_ref, b1_ref, w2t_ref, gmb_ref, seg_ref, out_ref,
          hbf_s, st_s, acc_s, rssum_s, pool_s):
    p = pl.program_id(0)
    i = pl.program_id(1)
    seg_row = seg_ref[0, :, :]
    g0, span_ok = _window(seg_ref)

    @pl.when(p == 0)
    def _phase_a():
        z = jnp.tanh(jnp.dot(h_ref[...], w1t_ref[...],
                             preferred_element_type=jnp.float32) + b1_ref[...])
        st = lax.dot_general(w2t_ref[...], z, (((0,), (1,)), ((), ())),
                             preferred_element_type=jnp.float32)
        st_s[i] = st
        hbf_s[i] = h_ref[...].astype(jnp.bfloat16)

        @pl.when(i == 0)
        def _():
            acc_s[...] = jnp.zeros((G, H), jnp.float32)

        e_t = jnp.exp(st - gmb_ref[...])

        @pl.when(span_ok)
        def _():
            ohT = _ohT(seg_row, g0, W)
            ps = lax.dot_general(e_t, ohT, (((1,), (1,)), ((), ())),
                                 preferred_element_type=jnp.float32)
            acc_s[pl.ds(g0, W), :] += jnp.transpose(ps)

        @pl.when(jnp.logical_not(span_ok))
        def _():
            ohT = _ohT(seg_row, 0, G)
            ps = lax.dot_general(e_t, ohT, (((1,), (1,)), ((), ())),
                                 preferred_element_type=jnp.float32)
            acc_s[...] += jnp.transpose(ps)

        @pl.when(i == NB - 1)
        def _():
            rssum_s[...] = 1.0 / (acc_s[...] + 1e-12)

    @pl.when(p == 1)
    def _phase_b():
        e_t = jnp.exp(st_s[i] - gmb_ref[...])
        hb = hbf_s[i]

        @pl.when(i == 0)
        def _():
            pool_s[...] = jnp.zeros((G, D_IN), jnp.float32)

        @pl.when(span_ok)
        def _():
            ohT = _ohT(seg_row, g0, W)
            rd_t = lax.dot_general(rssum_s[pl.ds(g0, W), :], ohT,
                                   (((0,), (0,)), ((), ())),
                                   preferred_element_type=jnp.float32)
            w_t = jnp.sum(e_t * rd_t, axis=0, keepdims=True) * (1.0 / H)
            ohTw = ohT.astype(jnp.bfloat16) * w_t.astype(jnp.bfloat16)
            pool_s[pl.ds(g0, W), :] += lax.dot_general(
                ohTw, hb, (((1,), (0,)), ((), ())),
                preferred_element_type=jnp.float32)

        @pl.when(jnp.logical_not(span_ok))
        def _():
            ohT = _ohT(seg_row, 0, G)
            rd_t = lax.dot_general(rssum_s[...], ohT,
                                   (((0,), (0,)), ((), ())),
                                   preferred_element_type=jnp.float32)
            w_t = jnp.sum(e_t * rd_t, axis=0, keepdims=True) * (1.0 / H)
            ohTw = ohT.astype(jnp.bfloat16) * w_t.astype(jnp.bfloat16)
            pool_s[...] += lax.dot_general(
                ohTw, hb, (((1,), (0,)), ((), ())),
                preferred_element_type=jnp.float32)

        @pl.when(i == NB - 1)
        def _():
            out_ref[...] = pool_s[...]


def kernel(h, segment_ids, fc1_w, fc1_b, fc2_w, fc2_b):
    seg3d = segment_ids.astype(jnp.int32).reshape(NB, 1, B)
    w1t = fc1_w.T
    w2t = fc2_w.T
    b1 = fc1_b.reshape(1, D_H)
    gmb = jnp.sum(jnp.abs(fc2_w), axis=1).reshape(H, 1)

    out = pl.pallas_call(
        _body,
        grid=(2, NB),
        in_specs=[
            pl.BlockSpec((B, D_IN), lambda p, i: (i * (1 - p), 0)),
            pl.BlockSpec((D_IN, D_H), lambda p, i: (0, 0)),
            pl.BlockSpec((1, D_H), lambda p, i: (0, 0)),
            pl.BlockSpec((D_H, H), lambda p, i: (0, 0)),
            pl.BlockSpec((H, 1), lambda p, i: (0, 0)),
            pl.BlockSpec((1, 1, B), lambda p, i: (i, 0, 0)),
        ],
        out_specs=pl.BlockSpec((G, D_IN), lambda p, i: (0, 0)),
        out_shape=jax.ShapeDtypeStruct((G, D_IN), jnp.float32),
        scratch_shapes=[
            pltpu.VMEM((NB, B, D_IN), jnp.bfloat16),
            pltpu.VMEM((NB, H, B), jnp.float32),
            pltpu.VMEM((G, H), jnp.float32),
            pltpu.VMEM((G, H), jnp.float32),
            pltpu.VMEM((G, D_IN), jnp.float32),
        ],
    )(h, w1t, b1, w2t, gmb, seg3d)

    return out
